# baseline (speedup 1.0000x reference)
import functools

import jax
import jax.numpy as jnp
from jax import lax
from jax.experimental import pallas as pl
from jax.experimental.pallas import tpu as pltpu
from jax.experimental.pallas import tpu_sc as plsc

NC = 2
NS = 16
NW = NC * NS
LANES = 16

_f32 = jnp.float32


def _wid(c, s):
  return s * NC + c


def _make_gather_rows(V, D, B):
  b_per = B // NW
  CG = 64
  n_chunks = b_per // CG
  mesh = plsc.VectorSubcoreMesh(core_axis_name="c", subcore_axis_name="s")

  @functools.partial(
      pl.kernel,
      mesh=mesh,
      out_type=jax.ShapeDtypeStruct((B, D), _f32),
      scratch_types=[
          pltpu.VMEM((b_per,), jnp.int32),
          pltpu.VMEM((CG, D), _f32),
          pltpu.SemaphoreType.DMA,
      ],
  )
  def k(table_hbm, idx_hbm, out_hbm, idx_v, rows_v, sem):
    wid = _wid(lax.axis_index("c"), lax.axis_index("s"))
    base = wid * b_per
    pltpu.sync_copy(idx_hbm.at[pl.ds(base, b_per)], idx_v)

    @pl.loop(0, n_chunks)
    def _chunks(g):
      pltpu.async_copy(
          table_hbm.at[idx_v.at[pl.ds(g * CG, CG)]], rows_v, sem).wait()
      pltpu.sync_copy(rows_v, out_hbm.at[pl.ds(base + g * CG, CG)])

  return k


def _make_gather_bias(N, V, B):
  b_per = B // NW
  CG = 64
  n_chunks = b_per // CG
  mesh = plsc.VectorSubcoreMesh(core_axis_name="c", subcore_axis_name="s")

  @functools.partial(
      pl.kernel,
      mesh=mesh,
      out_type=jax.ShapeDtypeStruct((B,), _f32),
      scratch_types=[
          pltpu.VMEM((b_per,), jnp.int32),
          pltpu.VMEM((CG,), jnp.int32),
          pltpu.VMEM((CG,), _f32),
          pltpu.SemaphoreType.DMA,
      ],
  )
  def k(nid_hbm, bias_hbm, idx_hbm, out_hbm, idx_v, mid_v, val_v, sem):
    wid = _wid(lax.axis_index("c"), lax.axis_index("s"))
    base = wid * b_per
    pltpu.sync_copy(idx_hbm.at[pl.ds(base, b_per)], idx_v)

    @pl.loop(0, n_chunks)
    def _chunks(g):
      pltpu.async_copy(
          nid_hbm.at[idx_v.at[pl.ds(g * CG, CG)]], mid_v, sem).wait()
      pltpu.async_copy(bias_hbm.at[mid_v], val_v, sem).wait()
      pltpu.sync_copy(val_v, out_hbm.at[pl.ds(base + g * CG, CG)])

  return k


def _make_segsum(N, D, E):
  e_per = E // NW
  CH = 80
  n_chunks = e_per // CH
  nseg = 8
  stripe = (N // NS) // 8 * 8
  tail = N - NS * stripe
  mesh = plsc.VectorSubcoreMesh(core_axis_name="c", subcore_axis_name="s")

  @functools.partial(
      pl.kernel,
      mesh=mesh,
      out_type=(
          jax.ShapeDtypeStruct((NC, N, D), _f32),
          jax.ShapeDtypeStruct((NC, 1, N), _f32),
      ),
      scratch_types=[
          [pltpu.VMEM((CH,), jnp.int32)] * 3,
          [pltpu.VMEM((CH,), jnp.int32)] * 3,
          [pltpu.VMEM((CH,), _f32)] * 3,
          [pltpu.VMEM((CH, D), _f32)] * 3,
          pltpu.VMEM((N,), _f32),
          pltpu.VMEM_SHARED((N, D), _f32),
          pltpu.VMEM_SHARED((N,), _f32),
          pltpu.SemaphoreType.DMA,
          [pltpu.SemaphoreType.DMA] * 3,
      ],
  )
  def k(nfeat_hbm, src_hbm, dst_hbm, w_hbm, agg_hbm, ws_hbm,
        idx_s, idx_d, wv, rows, zv, acc_sh, ws_sh, gsem, ssem):
    c = lax.axis_index("c")
    s = lax.axis_index("s")
    wid = _wid(c, s)
    zeros16 = jnp.zeros((LANES,), _f32)

    @pl.loop(0, CH)
    def _zrows(i):
      for j in range(nseg):
        rows[0][i, pl.ds(j * LANES, LANES)] = zeros16

    @pl.loop(0, N // LANES)
    def _zv(i):
      zv[pl.ds(i * LANES, LANES)] = zeros16

    @pl.when(s == 0)
    def _zws():
      pltpu.sync_copy(zv, ws_sh)

    sbase = s * stripe
    full, rem = stripe // CH, stripe % CH
    for t in range(full):
      pltpu.sync_copy(rows[0], acc_sh.at[pl.ds(sbase + t * CH, CH)])
    if rem:
      pltpu.sync_copy(rows[0].at[pl.ds(0, rem)],
                      acc_sh.at[pl.ds(sbase + full * CH, rem)])
    if tail:
      @pl.when(s == NS - 1)
      def _ztail():
        pltpu.sync_copy(rows[0].at[pl.ds(0, tail)],
                        acc_sh.at[pl.ds(NS * stripe, tail)])
    plsc.subcore_barrier()

    ebase = wid * e_per

    def stage_process(g, b):
      base = ebase + g * CH
      pltpu.sync_copy(src_hbm.at[pl.ds(base, CH)], idx_s[b])
      cp_d = pltpu.async_copy(dst_hbm.at[pl.ds(base, CH)], idx_d[b], gsem)
      cp_w = pltpu.async_copy(w_hbm.at[pl.ds(base, CH)], wv[b], gsem)
      cp_g = pltpu.async_copy(nfeat_hbm.at[idx_s[b]], rows[b], gsem)
      cp_d.wait()
      cp_w.wait()
      cp_g.wait()

      @pl.loop(0, CH // LANES)
      def _scale(t):
        wv16 = wv[b][pl.ds(t * LANES, LANES)]
        for kk in range(LANES):
          wb = jnp.full((LANES,), wv16[kk], _f32)
          i = t * LANES + kk
          for j in range(nseg):
            sl = pl.ds(j * LANES, LANES)
            rows[b][i, sl] = rows[b][i, sl] * wb

      pltpu.async_copy(rows[b], acc_sh.at[idx_d[b]], ssem[b], add=True)
      pltpu.async_copy(wv[b], ws_sh.at[idx_d[b]], ssem[b], add=True)

    def drain(b):
      pltpu.make_async_copy(rows[b], acc_sh.at[idx_d[b]], ssem[b]).wait()
      pltpu.make_async_copy(wv[b], ws_sh.at[idx_d[b]], ssem[b]).wait()

    NB = 3
    n_quads = n_chunks // NB

    @pl.loop(0, n_quads)
    def _edges(kk):
      for b in range(NB):
        @pl.when(kk > 0)
        def _d():
          drain(b)
        stage_process(NB * kk + b, b)

    for r in range(n_chunks - NB * n_quads):
      drain(r)
      stage_process(NB * n_quads + r, r)
    for b in range(NB):
      drain(b)

    plsc.subcore_barrier()
    pltpu.sync_copy(acc_sh.at[pl.ds(sbase, stripe)],
                    agg_hbm.at[c, pl.ds(sbase, stripe)])
    if tail:
      @pl.when(s == NS - 1)
      def _otail():
        pltpu.sync_copy(acc_sh.at[pl.ds(NS * stripe, tail)],
                        agg_hbm.at[c, pl.ds(NS * stripe, tail)])
    @pl.when(s == 0)
    def _ows():
      pltpu.sync_copy(ws_sh, ws_hbm.at[c, 0])

  return k


_BR = 2000


def _dot(a, b):
  return jnp.dot(a, b, preferred_element_type=_f32)


def _tc_proj_kernel(x_ref, wf_ref, bf_ref, emb_ref, qw_ref, qb_ref,
                    h_ref, nf_ref):
  h = _dot(x_ref[...], wf_ref[...]) + bf_ref[...][None, :] + emb_ref[...]
  h_ref[...] = h
  nf_ref[...] = jnp.maximum(_dot(h, qw_ref[...]) + qb_ref[...][None, :], 0.0)


def _tc_conv_kernel(aggp_ref, wsp_ref, hdst_ref, wa_ref, wb_ref, b_ref,
                    qw_ref, qb_ref, z_ref, nf_ref):
  agg = aggp_ref[0] + aggp_ref[1]
  ws = jnp.maximum(jnp.sum(wsp_ref[...], axis=1), 1.0)
  z = jnp.maximum(
      _dot(agg / ws[:, None], wa_ref[...]) + _dot(hdst_ref[...], wb_ref[...])
      + b_ref[...][None, :], 0.0)
  z_ref[...] = z
  nf_ref[...] = jnp.maximum(_dot(z, qw_ref[...]) + qb_ref[...][None, :], 0.0)


def _tc_final_kernel(aggp_ref, wsp_ref, hdst_ref, wa_ref, wb_ref, b_ref,
                     hitem_ref, z_ref):
  agg = aggp_ref[0] + aggp_ref[1]
  ws = jnp.maximum(jnp.sum(wsp_ref[...], axis=1), 1.0)
  z1 = jnp.maximum(
      _dot(agg / ws[:, None], wa_ref[...]) + _dot(hdst_ref[...], wb_ref[...])
      + b_ref[...][None, :], 0.0)
  z = hitem_ref[...] + z1
  nrm = jnp.sqrt(jnp.sum(z * z, axis=1, keepdims=True))
  nrm = jnp.where(nrm == 0.0, 1.0, nrm)
  z_ref[...] = z / nrm


def _tc_score_kernel(zps_ref, zpd_ref, zns_ref, znd_ref,
                     bps_ref, bpd_ref, bns_ref, bnd_ref, out_ref):
  pos = jnp.sum(zps_ref[...] * zpd_ref[...], axis=1) + bps_ref[...] + bpd_ref[...]
  neg = jnp.sum(zns_ref[...] * znd_ref[...], axis=1) + bns_ref[...] + bnd_ref[...]
  out_ref[...] = jnp.maximum(neg - pos + 1.0, 0.0)


def _row_spec(d):
  return pl.BlockSpec((_BR, d), lambda i: (i, 0))


def _full2_spec(a, b):
  return pl.BlockSpec((a, b), lambda i: (0, 0))


def _vec_spec(d):
  return pl.BlockSpec((d,), lambda i: (0,))


def _rowvec_spec():
  return pl.BlockSpec((_BR,), lambda i: (i,))


def kernel(x_feat, node_ids, edge_src0, edge_dst0, edge_w0,
           edge_src1, edge_dst1, edge_w1,
           pos_src, pos_dst, neg_src, neg_dst,
           W_feat, b_feat, emb_id,
           Q0w, Q0b, W0w, W0b, Q1w, Q1b, W1w, W1b, bias):
  N, D = x_feat.shape
  V = emb_id.shape[0]
  E = edge_src0.shape[0]
  EP = pos_src.shape[0]
  grid = (N // _BR,)

  def pad_idx(idx, tot):
    return jnp.concatenate(
        [idx.astype(jnp.int32), jnp.zeros((tot - idx.shape[0],), jnp.int32)])

  BN = 10240
  emb_rows = _make_gather_rows(V, D, BN)(emb_id, pad_idx(node_ids, BN))[:N]

  h_item, nf0 = pl.pallas_call(
      _tc_proj_kernel,
      grid=grid,
      in_specs=[_row_spec(D), _full2_spec(D, D), _vec_spec(D), _row_spec(D),
                _full2_spec(D, D), _vec_spec(D)],
      out_specs=[_row_spec(D), _row_spec(D)],
      out_shape=[jax.ShapeDtypeStruct((N, D), _f32)] * 2,
  )(x_feat, W_feat, b_feat, emb_rows, Q0w, Q0b)

  segsum = _make_segsum(N, D, E)

  aggp0, wsp0 = segsum(nf0, edge_src0.astype(jnp.int32),
                       edge_dst0.astype(jnp.int32), edge_w0)
  z0, nf1 = pl.pallas_call(
      _tc_conv_kernel,
      grid=grid,
      in_specs=[pl.BlockSpec((NC, _BR, D), lambda i: (0, i, 0)),
                pl.BlockSpec((_BR, NC), lambda i: (i, 0)),
                _row_spec(D), _full2_spec(D, D), _full2_spec(D, D),
                _vec_spec(D), _full2_spec(D, D), _vec_spec(D)],
      out_specs=[_row_spec(D), _row_spec(D)],
      out_shape=[jax.ShapeDtypeStruct((N, D), _f32)] * 2,
  )(aggp0, wsp0[:, 0, :].T, h_item, W0w[:D], W0w[D:], W0b, Q1w, Q1b)

  aggp1, wsp1 = segsum(nf1, edge_src1.astype(jnp.int32),
                       edge_dst1.astype(jnp.int32), edge_w1)
  z = pl.pallas_call(
      _tc_final_kernel,
      grid=grid,
      in_specs=[pl.BlockSpec((NC, _BR, D), lambda i: (0, i, 0)),
                pl.BlockSpec((_BR, NC), lambda i: (i, 0)),
                _row_spec(D), _full2_spec(D, D), _full2_spec(D, D),
                _vec_spec(D), _row_spec(D)],
      out_specs=_row_spec(D),
      out_shape=jax.ShapeDtypeStruct((N, D), _f32),
  )(aggp1, wsp1[:, 0, :].T, z0, W1w[:D], W1w[D:], W1b, h_item)

  EPP = 10240
  idx_all = jnp.concatenate([pad_idx(pos_src, EPP), pad_idx(pos_dst, EPP),
                             pad_idx(neg_src, EPP), pad_idx(neg_dst, EPP)])
  rows_all = _make_gather_rows(N, D, 4 * EPP)(z, idx_all)
  b_all = _make_gather_bias(N, V, 4 * EPP)(
      node_ids.astype(jnp.int32), bias, idx_all)

  sl = [slice(k * EPP, k * EPP + EP) for k in range(4)]
  out = pl.pallas_call(
      _tc_score_kernel,
      out_shape=jax.ShapeDtypeStruct((EP,), _f32),
  )(rows_all[sl[0]], rows_all[sl[1]], rows_all[sl[2]], rows_all[sl[3]],
    b_all[sl[0]], b_all[sl[1]], b_all[sl[2]], b_all[sl[3]])
  return out


# speedup vs baseline: 1.0527x; 1.0527x over previous
import functools

import jax
import jax.numpy as jnp
from jax import lax
from jax.experimental import pallas as pl
from jax.experimental.pallas import tpu as pltpu
from jax.experimental.pallas import tpu_sc as plsc

NC = 2
NS = 16
NW = NC * NS
LANES = 16

_f32 = jnp.float32


def _wid(c, s):
  return s * NC + c


def _make_gather_rows(V, D, B):
  b_per = B // NW
  CG = 64
  n_chunks = b_per // CG
  mesh = plsc.VectorSubcoreMesh(core_axis_name="c", subcore_axis_name="s")

  @functools.partial(
      pl.kernel,
      mesh=mesh,
      out_type=jax.ShapeDtypeStruct((B, D), _f32),
      scratch_types=[
          pltpu.VMEM((b_per,), jnp.int32),
          pltpu.VMEM((CG, D), _f32),
          pltpu.SemaphoreType.DMA,
      ],
  )
  def k(table_hbm, idx_hbm, out_hbm, idx_v, rows_v, sem):
    wid = _wid(lax.axis_index("c"), lax.axis_index("s"))
    base = wid * b_per
    pltpu.sync_copy(idx_hbm.at[pl.ds(base, b_per)], idx_v)

    @pl.loop(0, n_chunks)
    def _chunks(g):
      pltpu.async_copy(
          table_hbm.at[idx_v.at[pl.ds(g * CG, CG)]], rows_v, sem).wait()
      pltpu.sync_copy(rows_v, out_hbm.at[pl.ds(base + g * CG, CG)])

  return k


def _make_gather_bias(N, V, B):
  b_per = B // NW
  CG = 64
  n_chunks = b_per // CG
  mesh = plsc.VectorSubcoreMesh(core_axis_name="c", subcore_axis_name="s")

  @functools.partial(
      pl.kernel,
      mesh=mesh,
      out_type=jax.ShapeDtypeStruct((B,), _f32),
      scratch_types=[
          pltpu.VMEM((b_per,), jnp.int32),
          pltpu.VMEM((CG,), jnp.int32),
          pltpu.VMEM((CG,), _f32),
          pltpu.SemaphoreType.DMA,
      ],
  )
  def k(nid_hbm, bias_hbm, idx_hbm, out_hbm, idx_v, mid_v, val_v, sem):
    wid = _wid(lax.axis_index("c"), lax.axis_index("s"))
    base = wid * b_per
    pltpu.sync_copy(idx_hbm.at[pl.ds(base, b_per)], idx_v)

    @pl.loop(0, n_chunks)
    def _chunks(g):
      pltpu.async_copy(
          nid_hbm.at[idx_v.at[pl.ds(g * CG, CG)]], mid_v, sem).wait()
      pltpu.async_copy(bias_hbm.at[mid_v], val_v, sem).wait()
      pltpu.sync_copy(val_v, out_hbm.at[pl.ds(base + g * CG, CG)])

  return k


def _make_score(N, D, BP):
  p_per = BP // NW
  CP = 64
  n_chunks = p_per // CP
  nseg = D // LANES
  mesh = plsc.VectorSubcoreMesh(core_axis_name="c", subcore_axis_name="s")

  @functools.partial(
      pl.kernel,
      mesh=mesh,
      out_type=(
          jax.ShapeDtypeStruct((BP,), _f32),
          jax.ShapeDtypeStruct((BP,), _f32),
      ),
      scratch_types=[
          [pltpu.VMEM((p_per,), jnp.int32)] * 4,
          [pltpu.VMEM((CP, D), _f32)] * 4,
          [pltpu.VMEM((p_per,), _f32)] * 2,
          pltpu.SemaphoreType.DMA,
      ],
  )
  def k(z_hbm, ps_hbm, pd_hbm, ns_hbm, nd_hbm, pos_hbm, neg_hbm,
        idx, rows, outv, sem):
    wid = _wid(lax.axis_index("c"), lax.axis_index("s"))
    base = wid * p_per
    for a, h in enumerate((ps_hbm, pd_hbm, ns_hbm, nd_hbm)):
      pltpu.sync_copy(h.at[pl.ds(base, p_per)], idx[a])

    @pl.loop(0, n_chunks)
    def _chunks(g):
      sl = pl.ds(g * CP, CP)
      cps = [pltpu.async_copy(z_hbm.at[idx[a].at[sl]], rows[a], sem)
             for a in range(4)]
      for cp in cps:
        cp.wait()

      def lanesum(v):
        parts = [v[kk] for kk in range(LANES)]
        while len(parts) > 1:
          parts = [parts[ii] + parts[ii + 1] for ii in range(0, len(parts), 2)]
        return parts[0]

      iot = lax.iota(jnp.int32, LANES)

      @pl.loop(0, CP // LANES)
      def _grp(t):
        osl = pl.ds(g * CP + t * LANES, LANES)
        for o in range(2):
          red = jnp.zeros((LANES,), _f32)
          for kk in range(LANES):
            i = t * LANES + kk
            acc = (rows[2 * o][i, pl.ds(0, LANES)]
                   * rows[2 * o + 1][i, pl.ds(0, LANES)])
            for j in range(1, nseg):
              jl = pl.ds(j * LANES, LANES)
              acc = acc + rows[2 * o][i, jl] * rows[2 * o + 1][i, jl]
            red = jnp.where(iot == kk, jnp.full((LANES,), lanesum(acc), _f32),
                            red)
          outv[o][osl] = red

    pltpu.sync_copy(outv[0], pos_hbm.at[pl.ds(base, p_per)])
    pltpu.sync_copy(outv[1], neg_hbm.at[pl.ds(base, p_per)])

  return k


def _make_segsum(N, D, E):
  e_per = E // NW
  CH = 80
  n_chunks = e_per // CH
  nseg = 8
  stripe = (N // NS) // 8 * 8
  tail = N - NS * stripe
  mesh = plsc.VectorSubcoreMesh(core_axis_name="c", subcore_axis_name="s")

  @functools.partial(
      pl.kernel,
      mesh=mesh,
      out_type=(
          jax.ShapeDtypeStruct((NC, N, D), _f32),
          jax.ShapeDtypeStruct((NC, 1, N), _f32),
      ),
      scratch_types=[
          [pltpu.VMEM((CH,), jnp.int32)] * 3,
          [pltpu.VMEM((CH,), jnp.int32)] * 3,
          [pltpu.VMEM((CH,), _f32)] * 3,
          [pltpu.VMEM((CH, D), _f32)] * 3,
          pltpu.VMEM((N,), _f32),
          pltpu.VMEM_SHARED((N, D), _f32),
          pltpu.VMEM_SHARED((N,), _f32),
          pltpu.SemaphoreType.DMA,
          [pltpu.SemaphoreType.DMA] * 3,
      ],
  )
  def k(nfeat_hbm, src_hbm, dst_hbm, w_hbm, agg_hbm, ws_hbm,
        idx_s, idx_d, wv, rows, zv, acc_sh, ws_sh, gsem, ssem):
    c = lax.axis_index("c")
    s = lax.axis_index("s")
    wid = _wid(c, s)
    zeros16 = jnp.zeros((LANES,), _f32)

    @pl.loop(0, CH)
    def _zrows(i):
      for j in range(nseg):
        rows[0][i, pl.ds(j * LANES, LANES)] = zeros16

    @pl.loop(0, N // LANES)
    def _zv(i):
      zv[pl.ds(i * LANES, LANES)] = zeros16

    @pl.when(s == 0)
    def _zws():
      pltpu.sync_copy(zv, ws_sh)

    sbase = s * stripe
    full, rem = stripe // CH, stripe % CH
    for t in range(full):
      pltpu.sync_copy(rows[0], acc_sh.at[pl.ds(sbase + t * CH, CH)])
    if rem:
      pltpu.sync_copy(rows[0].at[pl.ds(0, rem)],
                      acc_sh.at[pl.ds(sbase + full * CH, rem)])
    if tail:
      @pl.when(s == NS - 1)
      def _ztail():
        pltpu.sync_copy(rows[0].at[pl.ds(0, tail)],
                        acc_sh.at[pl.ds(NS * stripe, tail)])
    plsc.subcore_barrier()

    ebase = wid * e_per

    def stage_process(g, b):
      base = ebase + g * CH
      pltpu.sync_copy(src_hbm.at[pl.ds(base, CH)], idx_s[b])
      cp_d = pltpu.async_copy(dst_hbm.at[pl.ds(base, CH)], idx_d[b], gsem)
      cp_w = pltpu.async_copy(w_hbm.at[pl.ds(base, CH)], wv[b], gsem)
      cp_g = pltpu.async_copy(nfeat_hbm.at[idx_s[b]], rows[b], gsem)
      cp_d.wait()
      cp_w.wait()
      cp_g.wait()

      @pl.loop(0, CH // LANES)
      def _scale(t):
        wv16 = wv[b][pl.ds(t * LANES, LANES)]
        for kk in range(LANES):
          wb = jnp.full((LANES,), wv16[kk], _f32)
          i = t * LANES + kk
          for j in range(nseg):
            sl = pl.ds(j * LANES, LANES)
            rows[b][i, sl] = rows[b][i, sl] * wb

      pltpu.async_copy(rows[b], acc_sh.at[idx_d[b]], ssem[b], add=True)
      pltpu.async_copy(wv[b], ws_sh.at[idx_d[b]], ssem[b], add=True)

    def drain(b):
      pltpu.make_async_copy(rows[b], acc_sh.at[idx_d[b]], ssem[b]).wait()
      pltpu.make_async_copy(wv[b], ws_sh.at[idx_d[b]], ssem[b]).wait()

    NB = 3
    n_quads = n_chunks // NB

    @pl.loop(0, n_quads)
    def _edges(kk):
      for b in range(NB):
        @pl.when(kk > 0)
        def _d():
          drain(b)
        stage_process(NB * kk + b, b)

    for r in range(n_chunks - NB * n_quads):
      drain(r)
      stage_process(NB * n_quads + r, r)
    for b in range(NB):
      drain(b)

    plsc.subcore_barrier()
    pltpu.sync_copy(acc_sh.at[pl.ds(sbase, stripe)],
                    agg_hbm.at[c, pl.ds(sbase, stripe)])
    if tail:
      @pl.when(s == NS - 1)
      def _otail():
        pltpu.sync_copy(acc_sh.at[pl.ds(NS * stripe, tail)],
                        agg_hbm.at[c, pl.ds(NS * stripe, tail)])
    @pl.when(s == 0)
    def _ows():
      pltpu.sync_copy(ws_sh, ws_hbm.at[c, 0])

  return k


_BR = 2000


def _dot(a, b):
  return jnp.dot(a, b, preferred_element_type=_f32)


def _tc_proj_kernel(x_ref, wf_ref, bf_ref, emb_ref, qw_ref, qb_ref,
                    h_ref, nf_ref):
  h = _dot(x_ref[...], wf_ref[...]) + bf_ref[...][None, :] + emb_ref[...]
  h_ref[...] = h
  nf_ref[...] = jnp.maximum(_dot(h, qw_ref[...]) + qb_ref[...][None, :], 0.0)


def _tc_conv_kernel(aggp_ref, wsp_ref, hdst_ref, wa_ref, wb_ref, b_ref,
                    qw_ref, qb_ref, z_ref, nf_ref):
  agg = aggp_ref[0] + aggp_ref[1]
  ws = jnp.maximum(jnp.sum(wsp_ref[...], axis=1), 1.0)
  z = jnp.maximum(
      _dot(agg / ws[:, None], wa_ref[...]) + _dot(hdst_ref[...], wb_ref[...])
      + b_ref[...][None, :], 0.0)
  z_ref[...] = z
  nf_ref[...] = jnp.maximum(_dot(z, qw_ref[...]) + qb_ref[...][None, :], 0.0)


def _tc_final_kernel(aggp_ref, wsp_ref, hdst_ref, wa_ref, wb_ref, b_ref,
                     hitem_ref, z_ref):
  agg = aggp_ref[0] + aggp_ref[1]
  ws = jnp.maximum(jnp.sum(wsp_ref[...], axis=1), 1.0)
  z1 = jnp.maximum(
      _dot(agg / ws[:, None], wa_ref[...]) + _dot(hdst_ref[...], wb_ref[...])
      + b_ref[...][None, :], 0.0)
  z = hitem_ref[...] + z1
  nrm = jnp.sqrt(jnp.sum(z * z, axis=1, keepdims=True))
  nrm = jnp.where(nrm == 0.0, 1.0, nrm)
  z_ref[...] = z / nrm


def _tc_score_kernel(posr_ref, negr_ref,
                     bps_ref, bpd_ref, bns_ref, bnd_ref, out_ref):
  pos = posr_ref[...] + bps_ref[...] + bpd_ref[...]
  neg = negr_ref[...] + bns_ref[...] + bnd_ref[...]
  out_ref[...] = jnp.maximum(neg - pos + 1.0, 0.0)


def _row_spec(d):
  return pl.BlockSpec((_BR, d), lambda i: (i, 0))


def _full2_spec(a, b):
  return pl.BlockSpec((a, b), lambda i: (0, 0))


def _vec_spec(d):
  return pl.BlockSpec((d,), lambda i: (0,))


def _rowvec_spec():
  return pl.BlockSpec((_BR,), lambda i: (i,))


def kernel(x_feat, node_ids, edge_src0, edge_dst0, edge_w0,
           edge_src1, edge_dst1, edge_w1,
           pos_src, pos_dst, neg_src, neg_dst,
           W_feat, b_feat, emb_id,
           Q0w, Q0b, W0w, W0b, Q1w, Q1b, W1w, W1b, bias):
  N, D = x_feat.shape
  V = emb_id.shape[0]
  E = edge_src0.shape[0]
  EP = pos_src.shape[0]
  grid = (N // _BR,)

  def pad_idx(idx, tot):
    return jnp.concatenate(
        [idx.astype(jnp.int32), jnp.zeros((tot - idx.shape[0],), jnp.int32)])

  BN = 10240
  emb_rows = _make_gather_rows(V, D, BN)(emb_id, pad_idx(node_ids, BN))[:N]

  h_item, nf0 = pl.pallas_call(
      _tc_proj_kernel,
      grid=grid,
      in_specs=[_row_spec(D), _full2_spec(D, D), _vec_spec(D), _row_spec(D),
                _full2_spec(D, D), _vec_spec(D)],
      out_specs=[_row_spec(D), _row_spec(D)],
      out_shape=[jax.ShapeDtypeStruct((N, D), _f32)] * 2,
  )(x_feat, W_feat, b_feat, emb_rows, Q0w, Q0b)

  segsum = _make_segsum(N, D, E)

  aggp0, wsp0 = segsum(nf0, edge_src0.astype(jnp.int32),
                       edge_dst0.astype(jnp.int32), edge_w0)
  z0, nf1 = pl.pallas_call(
      _tc_conv_kernel,
      grid=grid,
      in_specs=[pl.BlockSpec((NC, _BR, D), lambda i: (0, i, 0)),
                pl.BlockSpec((_BR, NC), lambda i: (i, 0)),
                _row_spec(D), _full2_spec(D, D), _full2_spec(D, D),
                _vec_spec(D), _full2_spec(D, D), _vec_spec(D)],
      out_specs=[_row_spec(D), _row_spec(D)],
      out_shape=[jax.ShapeDtypeStruct((N, D), _f32)] * 2,
  )(aggp0, wsp0[:, 0, :].T, h_item, W0w[:D], W0w[D:], W0b, Q1w, Q1b)

  aggp1, wsp1 = segsum(nf1, edge_src1.astype(jnp.int32),
                       edge_dst1.astype(jnp.int32), edge_w1)
  z = pl.pallas_call(
      _tc_final_kernel,
      grid=grid,
      in_specs=[pl.BlockSpec((NC, _BR, D), lambda i: (0, i, 0)),
                pl.BlockSpec((_BR, NC), lambda i: (i, 0)),
                _row_spec(D), _full2_spec(D, D), _full2_spec(D, D),
                _vec_spec(D), _row_spec(D)],
      out_specs=_row_spec(D),
      out_shape=jax.ShapeDtypeStruct((N, D), _f32),
  )(aggp1, wsp1[:, 0, :].T, z0, W1w[:D], W1w[D:], W1b, h_item)

  EPP = 10240
  idx_all = jnp.concatenate([pad_idx(pos_src, EPP), pad_idx(pos_dst, EPP),
                             pad_idx(neg_src, EPP), pad_idx(neg_dst, EPP)])
  posr, negr = _make_score(N, D, EPP)(
      z, pad_idx(pos_src, EPP), pad_idx(pos_dst, EPP),
      pad_idx(neg_src, EPP), pad_idx(neg_dst, EPP))
  b_all = _make_gather_bias(N, V, 4 * EPP)(
      node_ids.astype(jnp.int32), bias, idx_all)

  sl = [slice(k * EPP, k * EPP + EP) for k in range(4)]
  out = pl.pallas_call(
      _tc_score_kernel,
      out_shape=jax.ShapeDtypeStruct((EP,), _f32),
  )(posr[:EP], negr[:EP],
    b_all[sl[0]], b_all[sl[1]], b_all[sl[2]], b_all[sl[3]])
  return out


# speedup vs baseline: 1.2493x; 1.1867x over previous
import functools

import jax
import jax.numpy as jnp
from jax import lax
from jax.experimental import pallas as pl
from jax.experimental.pallas import tpu as pltpu
from jax.experimental.pallas import tpu_sc as plsc

NC = 2
NS = 16
NW = NC * NS
LANES = 16

_f32 = jnp.float32


def _wid(c, s):
  return s * NC + c


def _make_gather_rows(V, D, B):
  b_per = B // NW
  CG = 64
  n_chunks = b_per // CG
  mesh = plsc.VectorSubcoreMesh(core_axis_name="c", subcore_axis_name="s")

  @functools.partial(
      pl.kernel,
      mesh=mesh,
      out_type=jax.ShapeDtypeStruct((B, D), _f32),
      scratch_types=[
          pltpu.VMEM((b_per,), jnp.int32),
          pltpu.VMEM((CG, D), _f32),
          pltpu.SemaphoreType.DMA,
      ],
  )
  def k(table_hbm, idx_hbm, out_hbm, idx_v, rows_v, sem):
    wid = _wid(lax.axis_index("c"), lax.axis_index("s"))
    base = wid * b_per
    pltpu.sync_copy(idx_hbm.at[pl.ds(base, b_per)], idx_v)

    @pl.loop(0, n_chunks)
    def _chunks(g):
      pltpu.async_copy(
          table_hbm.at[idx_v.at[pl.ds(g * CG, CG)]], rows_v, sem).wait()
      pltpu.sync_copy(rows_v, out_hbm.at[pl.ds(base + g * CG, CG)])

  return k


def _make_gather_bias(N, V, B):
  b_per = B // NW
  CG = 128
  n_chunks = b_per // CG
  mesh = plsc.VectorSubcoreMesh(core_axis_name="c", subcore_axis_name="s")

  @functools.partial(
      pl.kernel,
      mesh=mesh,
      out_type=jax.ShapeDtypeStruct((B,), _f32),
      scratch_types=[
          pltpu.VMEM((b_per,), jnp.int32),
          [pltpu.VMEM((CG,), jnp.int32)] * 2,
          [pltpu.VMEM((CG,), _f32)] * 2,
          [pltpu.SemaphoreType.DMA] * 2,
          pltpu.SemaphoreType.DMA,
      ],
  )
  def k(nid_hbm, bias_hbm, idx_hbm, out_hbm, idx_v, mid_v, val_v, hsem, osem):
    wid = _wid(lax.axis_index("c"), lax.axis_index("s"))
    base = wid * b_per
    pltpu.sync_copy(idx_hbm.at[pl.ds(base, b_per)], idx_v)

    def hop1(g, b):
      pltpu.async_copy(nid_hbm.at[idx_v.at[pl.ds(g * CG, CG)]], mid_v[b],
                       hsem[b])

    def hop2_out(g, b):
      pltpu.make_async_copy(nid_hbm.at[idx_v.at[pl.ds(g * CG, CG)]],
                            mid_v[b], hsem[b]).wait()
      pltpu.async_copy(bias_hbm.at[mid_v[b]], val_v[b], osem).wait()
      pltpu.sync_copy(val_v[b], out_hbm.at[pl.ds(base + g * CG, CG)])

    assert n_chunks % 2 == 0
    hop1(0, 0)

    @pl.loop(0, n_chunks // 2)
    def _chunks(h):
      g = 2 * h
      hop1(g + 1, 1)
      hop2_out(g, 0)

      @pl.when(h < n_chunks // 2 - 1)
      def _p0():
        hop1(g + 2, 0)
      hop2_out(g + 1, 1)

  return k


def _make_score(N, D, BP):
  p_per = BP // NW
  CP = 64
  n_chunks = p_per // CP
  nseg = D // LANES
  mesh = plsc.VectorSubcoreMesh(core_axis_name="c", subcore_axis_name="s")

  @functools.partial(
      pl.kernel,
      mesh=mesh,
      out_type=(
          jax.ShapeDtypeStruct((BP,), _f32),
          jax.ShapeDtypeStruct((BP,), _f32),
      ),
      scratch_types=[
          [pltpu.VMEM((p_per,), jnp.int32)] * 4,
          [pltpu.VMEM((CP, D), _f32)] * 4,
          [pltpu.VMEM((p_per,), _f32)] * 2,
          pltpu.SemaphoreType.DMA,
      ],
  )
  def k(z_hbm, ps_hbm, pd_hbm, ns_hbm, nd_hbm, pos_hbm, neg_hbm,
        idx, rows, outv, sem):
    wid = _wid(lax.axis_index("c"), lax.axis_index("s"))
    base = wid * p_per
    for a, h in enumerate((ps_hbm, pd_hbm, ns_hbm, nd_hbm)):
      pltpu.sync_copy(h.at[pl.ds(base, p_per)], idx[a])

    @pl.loop(0, n_chunks)
    def _chunks(g):
      sl = pl.ds(g * CP, CP)
      cps = [pltpu.async_copy(z_hbm.at[idx[a].at[sl]], rows[a], sem)
             for a in range(4)]
      for cp in cps:
        cp.wait()

      def lanesum(v):
        parts = [v[kk] for kk in range(LANES)]
        while len(parts) > 1:
          parts = [parts[ii] + parts[ii + 1] for ii in range(0, len(parts), 2)]
        return parts[0]

      iot = lax.iota(jnp.int32, LANES)

      @pl.loop(0, CP // LANES)
      def _grp(t):
        osl = pl.ds(g * CP + t * LANES, LANES)
        for o in range(2):
          red = jnp.zeros((LANES,), _f32)
          for kk in range(LANES):
            i = t * LANES + kk
            acc = (rows[2 * o][i, pl.ds(0, LANES)]
                   * rows[2 * o + 1][i, pl.ds(0, LANES)])
            for j in range(1, nseg):
              jl = pl.ds(j * LANES, LANES)
              acc = acc + rows[2 * o][i, jl] * rows[2 * o + 1][i, jl]
            red = jnp.where(iot == kk, jnp.full((LANES,), lanesum(acc), _f32),
                            red)
          outv[o][osl] = red

    pltpu.sync_copy(outv[0], pos_hbm.at[pl.ds(base, p_per)])
    pltpu.sync_copy(outv[1], neg_hbm.at[pl.ds(base, p_per)])

  return k


def _make_segsum(N, D, E):
  e_per = E // NW
  CH = 128
  n_full = e_per // CH
  ch_tail = e_per - n_full * CH
  nseg = 8
  stripe = (N // NS) // 8 * 8
  tail = N - NS * stripe
  mesh = plsc.VectorSubcoreMesh(core_axis_name="c", subcore_axis_name="s")

  @functools.partial(
      pl.kernel,
      mesh=mesh,
      out_type=(
          jax.ShapeDtypeStruct((NC, N, D), _f32),
          jax.ShapeDtypeStruct((NC, 1, N), _f32),
      ),
      scratch_types=[
          [pltpu.VMEM((CH,), jnp.int32)] * 2,
          [pltpu.VMEM((CH,), jnp.int32)] * 2,
          [pltpu.VMEM((CH,), _f32)] * 2,
          [pltpu.VMEM((CH, D), _f32)] * 2,
          pltpu.VMEM((N,), _f32),
          pltpu.VMEM_SHARED((N, D), _f32),
          pltpu.VMEM_SHARED((N,), _f32),
          pltpu.SemaphoreType.DMA,
          [pltpu.SemaphoreType.DMA] * 2,
      ],
  )
  def k(nfeat_hbm, src_hbm, dst_hbm, w_hbm, agg_hbm, ws_hbm,
        idx_s, idx_d, wv, rows, zv, acc_sh, ws_sh, gsem, ssem):
    c = lax.axis_index("c")
    s = lax.axis_index("s")
    wid = _wid(c, s)
    zeros16 = jnp.zeros((LANES,), _f32)

    @pl.loop(0, CH)
    def _zrows(i):
      for j in range(nseg):
        rows[0][i, pl.ds(j * LANES, LANES)] = zeros16

    @pl.loop(0, N // LANES)
    def _zv(i):
      zv[pl.ds(i * LANES, LANES)] = zeros16

    @pl.when(s == 0)
    def _zws():
      pltpu.sync_copy(zv, ws_sh)

    sbase = s * stripe
    full, rem = stripe // CH, stripe % CH
    for t in range(full):
      pltpu.sync_copy(rows[0], acc_sh.at[pl.ds(sbase + t * CH, CH)])
    if rem:
      pltpu.sync_copy(rows[0].at[pl.ds(0, rem)],
                      acc_sh.at[pl.ds(sbase + full * CH, rem)])
    if tail:
      @pl.when(s == NS - 1)
      def _ztail():
        pltpu.sync_copy(rows[0].at[pl.ds(0, tail)],
                        acc_sh.at[pl.ds(NS * stripe, tail)])
    plsc.subcore_barrier()

    ebase = wid * e_per

    def stage_process(base, b, ch):
      csl = pl.ds(0, ch)
      pltpu.sync_copy(src_hbm.at[pl.ds(base, ch)], idx_s[b].at[csl])
      cp_d = pltpu.async_copy(dst_hbm.at[pl.ds(base, ch)], idx_d[b].at[csl],
                              gsem)
      cp_w = pltpu.async_copy(w_hbm.at[pl.ds(base, ch)], wv[b].at[csl], gsem)
      cp_g = pltpu.async_copy(nfeat_hbm.at[idx_s[b].at[csl]],
                              rows[b].at[csl], gsem)
      cp_d.wait()
      cp_w.wait()
      cp_g.wait()

      @pl.loop(0, ch // LANES)
      def _scale(t):
        wv16 = wv[b][pl.ds(t * LANES, LANES)]
        for kk in range(LANES):
          wb = jnp.full((LANES,), wv16[kk], _f32)
          i = t * LANES + kk
          for j in range(nseg):
            sl = pl.ds(j * LANES, LANES)
            rows[b][i, sl] = rows[b][i, sl] * wb

      pltpu.async_copy(rows[b].at[csl], acc_sh.at[idx_d[b].at[csl]],
                       ssem[b], add=True)
      pltpu.async_copy(wv[b].at[csl], ws_sh.at[idx_d[b].at[csl]],
                       ssem[b], add=True)

    def drain(b, ch):
      csl = pl.ds(0, ch)
      pltpu.make_async_copy(rows[b].at[csl], acc_sh.at[idx_d[b].at[csl]],
                            ssem[b]).wait()
      pltpu.make_async_copy(wv[b].at[csl], ws_sh.at[idx_d[b].at[csl]],
                            ssem[b]).wait()

    n_pairs = n_full // 2

    @pl.loop(0, n_pairs)
    def _edges(kk):
      for b in range(2):
        @pl.when(kk > 0)
        def _d():
          drain(b, CH)
        stage_process(ebase + (2 * kk + b) * CH, b, CH)

    drain(0, CH)
    drain(1, CH)
    if ch_tail:
      stage_process(ebase + n_full * CH, 0, ch_tail)
      drain(0, ch_tail)

    plsc.subcore_barrier()
    pltpu.sync_copy(acc_sh.at[pl.ds(sbase, stripe)],
                    agg_hbm.at[c, pl.ds(sbase, stripe)])
    if tail:
      @pl.when(s == NS - 1)
      def _otail():
        pltpu.sync_copy(acc_sh.at[pl.ds(NS * stripe, tail)],
                        agg_hbm.at[c, pl.ds(NS * stripe, tail)])
    @pl.when(s == 0)
    def _ows():
      pltpu.sync_copy(ws_sh, ws_hbm.at[c, 0])

  return k


_BR = 2000


def _dot(a, b):
  return jnp.dot(a, b, preferred_element_type=_f32)


def _tc_proj_kernel(x_ref, wf_ref, bf_ref, emb_ref, qw_ref, qb_ref,
                    h_ref, nf_ref):
  h = _dot(x_ref[...], wf_ref[...]) + bf_ref[...][None, :] + emb_ref[...]
  h_ref[...] = h
  nf_ref[...] = jnp.maximum(_dot(h, qw_ref[...]) + qb_ref[...][None, :], 0.0)


def _tc_conv_kernel(aggp_ref, wsp_ref, hdst_ref, wa_ref, wb_ref, b_ref,
                    qw_ref, qb_ref, z_ref, nf_ref):
  agg = aggp_ref[0] + aggp_ref[1]
  ws = jnp.maximum(jnp.sum(wsp_ref[...], axis=1), 1.0)
  z = jnp.maximum(
      _dot(agg / ws[:, None], wa_ref[...]) + _dot(hdst_ref[...], wb_ref[...])
      + b_ref[...][None, :], 0.0)
  z_ref[...] = z
  nf_ref[...] = jnp.maximum(_dot(z, qw_ref[...]) + qb_ref[...][None, :], 0.0)


def _tc_final_kernel(aggp_ref, wsp_ref, hdst_ref, wa_ref, wb_ref, b_ref,
                     hitem_ref, z_ref):
  agg = aggp_ref[0] + aggp_ref[1]
  ws = jnp.maximum(jnp.sum(wsp_ref[...], axis=1), 1.0)
  z1 = jnp.maximum(
      _dot(agg / ws[:, None], wa_ref[...]) + _dot(hdst_ref[...], wb_ref[...])
      + b_ref[...][None, :], 0.0)
  z = hitem_ref[...] + z1
  nrm = jnp.sqrt(jnp.sum(z * z, axis=1, keepdims=True))
  nrm = jnp.where(nrm == 0.0, 1.0, nrm)
  z_ref[...] = z / nrm


def _tc_score_kernel(posr_ref, negr_ref,
                     bps_ref, bpd_ref, bns_ref, bnd_ref, out_ref):
  pos = posr_ref[...] + bps_ref[...] + bpd_ref[...]
  neg = negr_ref[...] + bns_ref[...] + bnd_ref[...]
  out_ref[...] = jnp.maximum(neg - pos + 1.0, 0.0)


def _row_spec(d):
  return pl.BlockSpec((_BR, d), lambda i: (i, 0))


def _full2_spec(a, b):
  return pl.BlockSpec((a, b), lambda i: (0, 0))


def _vec_spec(d):
  return pl.BlockSpec((d,), lambda i: (0,))


def _rowvec_spec():
  return pl.BlockSpec((_BR,), lambda i: (i,))


def kernel(x_feat, node_ids, edge_src0, edge_dst0, edge_w0,
           edge_src1, edge_dst1, edge_w1,
           pos_src, pos_dst, neg_src, neg_dst,
           W_feat, b_feat, emb_id,
           Q0w, Q0b, W0w, W0b, Q1w, Q1b, W1w, W1b, bias):
  N, D = x_feat.shape
  V = emb_id.shape[0]
  E = edge_src0.shape[0]
  EP = pos_src.shape[0]
  grid = (N // _BR,)

  def pad_idx(idx, tot):
    return jnp.concatenate(
        [idx.astype(jnp.int32), jnp.zeros((tot - idx.shape[0],), jnp.int32)])

  BN = 10240
  emb_rows = _make_gather_rows(V, D, BN)(emb_id, pad_idx(node_ids, BN))[:N]

  h_item, nf0 = pl.pallas_call(
      _tc_proj_kernel,
      grid=grid,
      in_specs=[_row_spec(D), _full2_spec(D, D), _vec_spec(D), _row_spec(D),
                _full2_spec(D, D), _vec_spec(D)],
      out_specs=[_row_spec(D), _row_spec(D)],
      out_shape=[jax.ShapeDtypeStruct((N, D), _f32)] * 2,
  )(x_feat, W_feat, b_feat, emb_rows, Q0w, Q0b)

  segsum = _make_segsum(N, D, E)

  aggp0, wsp0 = segsum(nf0, edge_src0.astype(jnp.int32),
                       edge_dst0.astype(jnp.int32), edge_w0)
  z0, nf1 = pl.pallas_call(
      _tc_conv_kernel,
      grid=grid,
      in_specs=[pl.BlockSpec((NC, _BR, D), lambda i: (0, i, 0)),
                pl.BlockSpec((_BR, NC), lambda i: (i, 0)),
                _row_spec(D), _full2_spec(D, D), _full2_spec(D, D),
                _vec_spec(D), _full2_spec(D, D), _vec_spec(D)],
      out_specs=[_row_spec(D), _row_spec(D)],
      out_shape=[jax.ShapeDtypeStruct((N, D), _f32)] * 2,
  )(aggp0, wsp0[:, 0, :].T, h_item, W0w[:D], W0w[D:], W0b, Q1w, Q1b)

  aggp1, wsp1 = segsum(nf1, edge_src1.astype(jnp.int32),
                       edge_dst1.astype(jnp.int32), edge_w1)
  z = pl.pallas_call(
      _tc_final_kernel,
      grid=grid,
      in_specs=[pl.BlockSpec((NC, _BR, D), lambda i: (0, i, 0)),
                pl.BlockSpec((_BR, NC), lambda i: (i, 0)),
                _row_spec(D), _full2_spec(D, D), _full2_spec(D, D),
                _vec_spec(D), _row_spec(D)],
      out_specs=_row_spec(D),
      out_shape=jax.ShapeDtypeStruct((N, D), _f32),
  )(aggp1, wsp1[:, 0, :].T, z0, W1w[:D], W1w[D:], W1b, h_item)

  EPP = 10240
  idx_all = jnp.concatenate([pad_idx(pos_src, EPP), pad_idx(pos_dst, EPP),
                             pad_idx(neg_src, EPP), pad_idx(neg_dst, EPP)])
  posr, negr = _make_score(N, D, EPP)(
      z, pad_idx(pos_src, EPP), pad_idx(pos_dst, EPP),
      pad_idx(neg_src, EPP), pad_idx(neg_dst, EPP))
  b_all = _make_gather_bias(N, V, 4 * EPP)(
      node_ids.astype(jnp.int32), bias, idx_all)

  sl = [slice(k * EPP, k * EPP + EP) for k in range(4)]
  out = pl.pallas_call(
      _tc_score_kernel,
      out_shape=jax.ShapeDtypeStruct((EP,), _f32),
  )(posr[:EP], negr[:EP],
    b_all[sl[0]], b_all[sl[1]], b_all[sl[2]], b_all[sl[3]])
  return out


# speedup vs baseline: 1.2518x; 1.0020x over previous
import functools

import jax
import jax.numpy as jnp
from jax import lax
from jax.experimental import pallas as pl
from jax.experimental.pallas import tpu as pltpu
from jax.experimental.pallas import tpu_sc as plsc

NC = 2
NS = 16
NW = NC * NS
LANES = 16

_f32 = jnp.float32


def _wid(c, s):
  return s * NC + c


def _make_gather_rows(V, D, B):
  b_per = B // NW
  CG = 64
  n_chunks = b_per // CG
  mesh = plsc.VectorSubcoreMesh(core_axis_name="c", subcore_axis_name="s")

  @functools.partial(
      pl.kernel,
      mesh=mesh,
      out_type=jax.ShapeDtypeStruct((B, D), _f32),
      scratch_types=[
          pltpu.VMEM((b_per,), jnp.int32),
          [pltpu.VMEM((CG, D), _f32)] * 2,
          [pltpu.SemaphoreType.DMA] * 2,
      ],
  )
  def k(table_hbm, idx_hbm, out_hbm, idx_v, rows_v, sem):
    wid = _wid(lax.axis_index("c"), lax.axis_index("s"))
    base = wid * b_per
    pltpu.sync_copy(idx_hbm.at[pl.ds(base, b_per)], idx_v)

    def fire(g, b):
      pltpu.async_copy(
          table_hbm.at[idx_v.at[pl.ds(g * CG, CG)]], rows_v[b], sem[b])

    def flush(g, b):
      pltpu.make_async_copy(
          table_hbm.at[idx_v.at[pl.ds(g * CG, CG)]], rows_v[b], sem[b]).wait()
      pltpu.sync_copy(rows_v[b], out_hbm.at[pl.ds(base + g * CG, CG)])

    fire(0, 0)

    @pl.loop(0, n_chunks // 2)
    def _chunks(h):
      g = 2 * h
      fire(g + 1, 1)
      flush(g, 0)

      @pl.when(h < n_chunks // 2 - 1)
      def _p():
        fire(g + 2, 0)
      flush(g + 1, 1)

    if n_chunks % 2:
      g = n_chunks - 1
      fire(g, 0)
      flush(g, 0)

  return k


def _make_gather_bias(N, V, B):
  b_per = B // NW
  CG = 128
  n_chunks = b_per // CG
  mesh = plsc.VectorSubcoreMesh(core_axis_name="c", subcore_axis_name="s")

  @functools.partial(
      pl.kernel,
      mesh=mesh,
      out_type=jax.ShapeDtypeStruct((B,), _f32),
      scratch_types=[
          pltpu.VMEM((b_per,), jnp.int32),
          [pltpu.VMEM((CG,), jnp.int32)] * 2,
          [pltpu.VMEM((CG,), _f32)] * 2,
          [pltpu.SemaphoreType.DMA] * 2,
          pltpu.SemaphoreType.DMA,
      ],
  )
  def k(nid_hbm, bias_hbm, idx_hbm, out_hbm, idx_v, mid_v, val_v, hsem, osem):
    wid = _wid(lax.axis_index("c"), lax.axis_index("s"))
    base = wid * b_per
    pltpu.sync_copy(idx_hbm.at[pl.ds(base, b_per)], idx_v)

    def hop1(g, b):
      pltpu.async_copy(nid_hbm.at[idx_v.at[pl.ds(g * CG, CG)]], mid_v[b],
                       hsem[b])

    def hop2_out(g, b):
      pltpu.make_async_copy(nid_hbm.at[idx_v.at[pl.ds(g * CG, CG)]],
                            mid_v[b], hsem[b]).wait()
      pltpu.async_copy(bias_hbm.at[mid_v[b]], val_v[b], osem).wait()
      pltpu.sync_copy(val_v[b], out_hbm.at[pl.ds(base + g * CG, CG)])

    assert n_chunks % 2 == 0
    hop1(0, 0)

    @pl.loop(0, n_chunks // 2)
    def _chunks(h):
      g = 2 * h
      hop1(g + 1, 1)
      hop2_out(g, 0)

      @pl.when(h < n_chunks // 2 - 1)
      def _p0():
        hop1(g + 2, 0)
      hop2_out(g + 1, 1)

  return k


def _make_score(N, D, BP):
  p_per = BP // NW
  CP = 64
  n_chunks = p_per // CP
  nseg = D // LANES
  mesh = plsc.VectorSubcoreMesh(core_axis_name="c", subcore_axis_name="s")

  @functools.partial(
      pl.kernel,
      mesh=mesh,
      out_type=(
          jax.ShapeDtypeStruct((BP,), _f32),
          jax.ShapeDtypeStruct((BP,), _f32),
      ),
      scratch_types=[
          [pltpu.VMEM((p_per,), jnp.int32)] * 4,
          [pltpu.VMEM((CP, D), _f32)] * 4,
          [pltpu.VMEM((p_per,), _f32)] * 2,
          pltpu.SemaphoreType.DMA,
      ],
  )
  def k(z_hbm, ps_hbm, pd_hbm, ns_hbm, nd_hbm, pos_hbm, neg_hbm,
        idx, rows, outv, sem):
    wid = _wid(lax.axis_index("c"), lax.axis_index("s"))
    base = wid * p_per
    for a, h in enumerate((ps_hbm, pd_hbm, ns_hbm, nd_hbm)):
      pltpu.sync_copy(h.at[pl.ds(base, p_per)], idx[a])

    @pl.loop(0, n_chunks)
    def _chunks(g):
      sl = pl.ds(g * CP, CP)
      cps = [pltpu.async_copy(z_hbm.at[idx[a].at[sl]], rows[a], sem)
             for a in range(4)]
      for cp in cps:
        cp.wait()

      iot = lax.iota(jnp.int32, LANES)

      def lanesum(v):
        for sh in (8, 4, 2, 1):
          v = v + v.at[jnp.bitwise_xor(iot, sh)].get(
              mode="promise_in_bounds")
        return v

      @pl.loop(0, CP // LANES)
      def _grp(t):
        osl = pl.ds(g * CP + t * LANES, LANES)
        for o in range(2):
          red = jnp.zeros((LANES,), _f32)
          for kk in range(LANES):
            i = t * LANES + kk
            acc = (rows[2 * o][i, pl.ds(0, LANES)]
                   * rows[2 * o + 1][i, pl.ds(0, LANES)])
            for j in range(1, nseg):
              jl = pl.ds(j * LANES, LANES)
              acc = acc + rows[2 * o][i, jl] * rows[2 * o + 1][i, jl]
            red = jnp.where(iot == kk, lanesum(acc), red)
          outv[o][osl] = red

    pltpu.sync_copy(outv[0], pos_hbm.at[pl.ds(base, p_per)])
    pltpu.sync_copy(outv[1], neg_hbm.at[pl.ds(base, p_per)])

  return k


def _make_segsum(N, D, E):
  e_per = E // NW
  CH = 128
  n_full = e_per // CH
  ch_tail = e_per - n_full * CH
  nseg = 8
  stripe = (N // NS) // 8 * 8
  tail = N - NS * stripe
  mesh = plsc.VectorSubcoreMesh(core_axis_name="c", subcore_axis_name="s")

  @functools.partial(
      pl.kernel,
      mesh=mesh,
      out_type=(
          jax.ShapeDtypeStruct((NC, N, D), _f32),
          jax.ShapeDtypeStruct((NC, 1, N), _f32),
      ),
      scratch_types=[
          [pltpu.VMEM((CH,), jnp.int32)] * 2,
          [pltpu.VMEM((CH,), jnp.int32)] * 2,
          [pltpu.VMEM((CH,), _f32)] * 2,
          [pltpu.VMEM((CH, D), _f32)] * 2,
          pltpu.VMEM((N,), _f32),
          pltpu.VMEM_SHARED((N, D), _f32),
          pltpu.VMEM_SHARED((N,), _f32),
          pltpu.SemaphoreType.DMA,
          [pltpu.SemaphoreType.DMA] * 2,
      ],
  )
  def k(nfeat_hbm, src_hbm, dst_hbm, w_hbm, agg_hbm, ws_hbm,
        idx_s, idx_d, wv, rows, zv, acc_sh, ws_sh, gsem, ssem):
    c = lax.axis_index("c")
    s = lax.axis_index("s")
    wid = _wid(c, s)
    zeros16 = jnp.zeros((LANES,), _f32)

    @pl.loop(0, CH)
    def _zrows(i):
      for j in range(nseg):
        rows[0][i, pl.ds(j * LANES, LANES)] = zeros16

    @pl.loop(0, N // LANES)
    def _zv(i):
      zv[pl.ds(i * LANES, LANES)] = zeros16

    @pl.when(s == 0)
    def _zws():
      pltpu.sync_copy(zv, ws_sh)

    sbase = s * stripe
    full, rem = stripe // CH, stripe % CH
    for t in range(full):
      pltpu.sync_copy(rows[0], acc_sh.at[pl.ds(sbase + t * CH, CH)])
    if rem:
      pltpu.sync_copy(rows[0].at[pl.ds(0, rem)],
                      acc_sh.at[pl.ds(sbase + full * CH, rem)])
    if tail:
      @pl.when(s == NS - 1)
      def _ztail():
        pltpu.sync_copy(rows[0].at[pl.ds(0, tail)],
                        acc_sh.at[pl.ds(NS * stripe, tail)])
    plsc.subcore_barrier()

    ebase = wid * e_per

    def stage_process(base, b, ch):
      csl = pl.ds(0, ch)
      pltpu.sync_copy(src_hbm.at[pl.ds(base, ch)], idx_s[b].at[csl])
      cp_d = pltpu.async_copy(dst_hbm.at[pl.ds(base, ch)], idx_d[b].at[csl],
                              gsem)
      cp_w = pltpu.async_copy(w_hbm.at[pl.ds(base, ch)], wv[b].at[csl], gsem)
      cp_g = pltpu.async_copy(nfeat_hbm.at[idx_s[b].at[csl]],
                              rows[b].at[csl], gsem)
      cp_d.wait()
      cp_w.wait()
      cp_g.wait()

      @pl.loop(0, ch // LANES)
      def _scale(t):
        wv16 = wv[b][pl.ds(t * LANES, LANES)]
        for kk in range(LANES):
          wb = jnp.full((LANES,), wv16[kk], _f32)
          i = t * LANES + kk
          for j in range(nseg):
            sl = pl.ds(j * LANES, LANES)
            rows[b][i, sl] = rows[b][i, sl] * wb

      pltpu.async_copy(rows[b].at[csl], acc_sh.at[idx_d[b].at[csl]],
                       ssem[b], add=True)
      pltpu.async_copy(wv[b].at[csl], ws_sh.at[idx_d[b].at[csl]],
                       ssem[b], add=True)

    def drain(b, ch):
      csl = pl.ds(0, ch)
      pltpu.make_async_copy(rows[b].at[csl], acc_sh.at[idx_d[b].at[csl]],
                            ssem[b]).wait()
      pltpu.make_async_copy(wv[b].at[csl], ws_sh.at[idx_d[b].at[csl]],
                            ssem[b]).wait()

    n_pairs = n_full // 2

    @pl.loop(0, n_pairs)
    def _edges(kk):
      for b in range(2):
        @pl.when(kk > 0)
        def _d():
          drain(b, CH)
        stage_process(ebase + (2 * kk + b) * CH, b, CH)

    drain(0, CH)
    drain(1, CH)
    if ch_tail:
      stage_process(ebase + n_full * CH, 0, ch_tail)
      drain(0, ch_tail)

    plsc.subcore_barrier()
    pltpu.sync_copy(acc_sh.at[pl.ds(sbase, stripe)],
                    agg_hbm.at[c, pl.ds(sbase, stripe)])
    if tail:
      @pl.when(s == NS - 1)
      def _otail():
        pltpu.sync_copy(acc_sh.at[pl.ds(NS * stripe, tail)],
                        agg_hbm.at[c, pl.ds(NS * stripe, tail)])
    @pl.when(s == 0)
    def _ows():
      pltpu.sync_copy(ws_sh, ws_hbm.at[c, 0])

  return k


_BR = 2000


def _dot(a, b):
  return jnp.dot(a, b, preferred_element_type=_f32)


def _tc_proj_kernel(x_ref, wf_ref, bf_ref, emb_ref, qw_ref, qb_ref,
                    h_ref, nf_ref):
  h = _dot(x_ref[...], wf_ref[...]) + bf_ref[...][None, :] + emb_ref[...]
  h_ref[...] = h
  nf_ref[...] = jnp.maximum(_dot(h, qw_ref[...]) + qb_ref[...][None, :], 0.0)


def _tc_conv_kernel(aggp_ref, wsp_ref, hdst_ref, wa_ref, wb_ref, b_ref,
                    qw_ref, qb_ref, z_ref, nf_ref):
  agg = aggp_ref[0] + aggp_ref[1]
  ws = jnp.maximum(jnp.sum(wsp_ref[...], axis=1), 1.0)
  z = jnp.maximum(
      _dot(agg / ws[:, None], wa_ref[...]) + _dot(hdst_ref[...], wb_ref[...])
      + b_ref[...][None, :], 0.0)
  z_ref[...] = z
  nf_ref[...] = jnp.maximum(_dot(z, qw_ref[...]) + qb_ref[...][None, :], 0.0)


def _tc_final_kernel(aggp_ref, wsp_ref, hdst_ref, wa_ref, wb_ref, b_ref,
                     hitem_ref, z_ref):
  agg = aggp_ref[0] + aggp_ref[1]
  ws = jnp.maximum(jnp.sum(wsp_ref[...], axis=1), 1.0)
  z1 = jnp.maximum(
      _dot(agg / ws[:, None], wa_ref[...]) + _dot(hdst_ref[...], wb_ref[...])
      + b_ref[...][None, :], 0.0)
  z = hitem_ref[...] + z1
  nrm = jnp.sqrt(jnp.sum(z * z, axis=1, keepdims=True))
  nrm = jnp.where(nrm == 0.0, 1.0, nrm)
  z_ref[...] = z / nrm


def _tc_score_kernel(posr_ref, negr_ref,
                     bps_ref, bpd_ref, bns_ref, bnd_ref, out_ref):
  pos = posr_ref[...] + bps_ref[...] + bpd_ref[...]
  neg = negr_ref[...] + bns_ref[...] + bnd_ref[...]
  out_ref[...] = jnp.maximum(neg - pos + 1.0, 0.0)


def _row_spec(d):
  return pl.BlockSpec((_BR, d), lambda i: (i, 0))


def _full2_spec(a, b):
  return pl.BlockSpec((a, b), lambda i: (0, 0))


def _vec_spec(d):
  return pl.BlockSpec((d,), lambda i: (0,))


def _rowvec_spec():
  return pl.BlockSpec((_BR,), lambda i: (i,))


def kernel(x_feat, node_ids, edge_src0, edge_dst0, edge_w0,
           edge_src1, edge_dst1, edge_w1,
           pos_src, pos_dst, neg_src, neg_dst,
           W_feat, b_feat, emb_id,
           Q0w, Q0b, W0w, W0b, Q1w, Q1b, W1w, W1b, bias):
  N, D = x_feat.shape
  V = emb_id.shape[0]
  E = edge_src0.shape[0]
  EP = pos_src.shape[0]
  grid = (N // _BR,)

  def pad_idx(idx, tot):
    return jnp.concatenate(
        [idx.astype(jnp.int32), jnp.zeros((tot - idx.shape[0],), jnp.int32)])

  BN = 10240
  emb_rows = _make_gather_rows(V, D, BN)(emb_id, pad_idx(node_ids, BN))[:N]

  h_item, nf0 = pl.pallas_call(
      _tc_proj_kernel,
      grid=grid,
      in_specs=[_row_spec(D), _full2_spec(D, D), _vec_spec(D), _row_spec(D),
                _full2_spec(D, D), _vec_spec(D)],
      out_specs=[_row_spec(D), _row_spec(D)],
      out_shape=[jax.ShapeDtypeStruct((N, D), _f32)] * 2,
  )(x_feat, W_feat, b_feat, emb_rows, Q0w, Q0b)

  segsum = _make_segsum(N, D, E)

  aggp0, wsp0 = segsum(nf0, edge_src0.astype(jnp.int32),
                       edge_dst0.astype(jnp.int32), edge_w0)
  z0, nf1 = pl.pallas_call(
      _tc_conv_kernel,
      grid=grid,
      in_specs=[pl.BlockSpec((NC, _BR, D), lambda i: (0, i, 0)),
                pl.BlockSpec((_BR, NC), lambda i: (i, 0)),
                _row_spec(D), _full2_spec(D, D), _full2_spec(D, D),
                _vec_spec(D), _full2_spec(D, D), _vec_spec(D)],
      out_specs=[_row_spec(D), _row_spec(D)],
      out_shape=[jax.ShapeDtypeStruct((N, D), _f32)] * 2,
  )(aggp0, wsp0[:, 0, :].T, h_item, W0w[:D], W0w[D:], W0b, Q1w, Q1b)

  aggp1, wsp1 = segsum(nf1, edge_src1.astype(jnp.int32),
                       edge_dst1.astype(jnp.int32), edge_w1)
  z = pl.pallas_call(
      _tc_final_kernel,
      grid=grid,
      in_specs=[pl.BlockSpec((NC, _BR, D), lambda i: (0, i, 0)),
                pl.BlockSpec((_BR, NC), lambda i: (i, 0)),
                _row_spec(D), _full2_spec(D, D), _full2_spec(D, D),
                _vec_spec(D), _row_spec(D)],
      out_specs=_row_spec(D),
      out_shape=jax.ShapeDtypeStruct((N, D), _f32),
  )(aggp1, wsp1[:, 0, :].T, z0, W1w[:D], W1w[D:], W1b, h_item)

  EPP = 10240
  idx_all = jnp.concatenate([pad_idx(pos_src, EPP), pad_idx(pos_dst, EPP),
                             pad_idx(neg_src, EPP), pad_idx(neg_dst, EPP)])
  posr, negr = _make_score(N, D, EPP)(
      z, pad_idx(pos_src, EPP), pad_idx(pos_dst, EPP),
      pad_idx(neg_src, EPP), pad_idx(neg_dst, EPP))
  b_all = _make_gather_bias(N, V, 4 * EPP)(
      node_ids.astype(jnp.int32), bias, idx_all)

  sl = [slice(k * EPP, k * EPP + EP) for k in range(4)]
  out = pl.pallas_call(
      _tc_score_kernel,
      out_shape=jax.ShapeDtypeStruct((EP,), _f32),
  )(posr[:EP], negr[:EP],
    b_all[sl[0]], b_all[sl[1]], b_all[sl[2]], b_all[sl[3]])
  return out


# speedup vs baseline: 1.2559x; 1.0033x over previous
import functools

import jax
import jax.numpy as jnp
from jax import lax
from jax.experimental import pallas as pl
from jax.experimental.pallas import tpu as pltpu
from jax.experimental.pallas import tpu_sc as plsc

NC = 2
NS = 16
NW = NC * NS
LANES = 16

_f32 = jnp.float32


def _wid(c, s):
  return s * NC + c


def _make_gather_rows(V, D, B):
  b_per = B // NW
  CG = 64
  n_chunks = b_per // CG
  mesh = plsc.VectorSubcoreMesh(core_axis_name="c", subcore_axis_name="s")

  @functools.partial(
      pl.kernel,
      mesh=mesh,
      out_type=jax.ShapeDtypeStruct((B, D), _f32),
      scratch_types=[
          pltpu.VMEM((b_per,), jnp.int32),
          [pltpu.VMEM((CG, D), _f32)] * 2,
          [pltpu.SemaphoreType.DMA] * 2,
      ],
  )
  def k(table_hbm, idx_hbm, out_hbm, idx_v, rows_v, sem):
    wid = _wid(lax.axis_index("c"), lax.axis_index("s"))
    base = wid * b_per
    pltpu.sync_copy(idx_hbm.at[pl.ds(base, b_per)], idx_v)

    def fire(g, b):
      pltpu.async_copy(
          table_hbm.at[idx_v.at[pl.ds(g * CG, CG)]], rows_v[b], sem[b])

    def flush(g, b):
      pltpu.make_async_copy(
          table_hbm.at[idx_v.at[pl.ds(g * CG, CG)]], rows_v[b], sem[b]).wait()
      pltpu.sync_copy(rows_v[b], out_hbm.at[pl.ds(base + g * CG, CG)])

    fire(0, 0)

    @pl.loop(0, n_chunks // 2)
    def _chunks(h):
      g = 2 * h
      fire(g + 1, 1)
      flush(g, 0)

      @pl.when(h < n_chunks // 2 - 1)
      def _p():
        fire(g + 2, 0)
      flush(g + 1, 1)

    if n_chunks % 2:
      g = n_chunks - 1
      fire(g, 0)
      flush(g, 0)

  return k


def _make_gather_bias(N, V, B):
  b_per = B // NW
  CG = 128
  n_chunks = b_per // CG
  mesh = plsc.VectorSubcoreMesh(core_axis_name="c", subcore_axis_name="s")

  @functools.partial(
      pl.kernel,
      mesh=mesh,
      out_type=jax.ShapeDtypeStruct((B,), _f32),
      scratch_types=[
          pltpu.VMEM((b_per,), jnp.int32),
          [pltpu.VMEM((CG,), jnp.int32)] * 2,
          [pltpu.VMEM((CG,), _f32)] * 2,
          [pltpu.SemaphoreType.DMA] * 2,
          pltpu.SemaphoreType.DMA,
      ],
  )
  def k(nid_hbm, bias_hbm, idx_hbm, out_hbm, idx_v, mid_v, val_v, hsem, osem):
    wid = _wid(lax.axis_index("c"), lax.axis_index("s"))
    base = wid * b_per
    pltpu.sync_copy(idx_hbm.at[pl.ds(base, b_per)], idx_v)

    def hop1(g, b):
      pltpu.async_copy(nid_hbm.at[idx_v.at[pl.ds(g * CG, CG)]], mid_v[b],
                       hsem[b])

    def hop2_out(g, b):
      pltpu.make_async_copy(nid_hbm.at[idx_v.at[pl.ds(g * CG, CG)]],
                            mid_v[b], hsem[b]).wait()
      pltpu.async_copy(bias_hbm.at[mid_v[b]], val_v[b], osem).wait()
      pltpu.sync_copy(val_v[b], out_hbm.at[pl.ds(base + g * CG, CG)])

    assert n_chunks % 2 == 0
    hop1(0, 0)

    @pl.loop(0, n_chunks // 2)
    def _chunks(h):
      g = 2 * h
      hop1(g + 1, 1)
      hop2_out(g, 0)

      @pl.when(h < n_chunks // 2 - 1)
      def _p0():
        hop1(g + 2, 0)
      hop2_out(g + 1, 1)

  return k


def _make_score(N, D, BP):
  p_per = BP // NW
  CP = 64
  n_chunks = p_per // CP
  nseg = D // LANES
  mesh = plsc.VectorSubcoreMesh(core_axis_name="c", subcore_axis_name="s")

  @functools.partial(
      pl.kernel,
      mesh=mesh,
      out_type=(
          jax.ShapeDtypeStruct((BP,), _f32),
          jax.ShapeDtypeStruct((BP,), _f32),
      ),
      scratch_types=[
          [pltpu.VMEM((p_per,), jnp.int32)] * 4,
          [pltpu.VMEM((CP, D), _f32)] * 4,
          [pltpu.VMEM((p_per,), _f32)] * 2,
          pltpu.SemaphoreType.DMA,
      ],
  )
  def k(z_hbm, ps_hbm, pd_hbm, ns_hbm, nd_hbm, pos_hbm, neg_hbm,
        idx, rows, outv, sem):
    wid = _wid(lax.axis_index("c"), lax.axis_index("s"))
    base = wid * p_per
    for a, h in enumerate((ps_hbm, pd_hbm, ns_hbm, nd_hbm)):
      pltpu.sync_copy(h.at[pl.ds(base, p_per)], idx[a])

    @pl.loop(0, n_chunks)
    def _chunks(g):
      sl = pl.ds(g * CP, CP)
      cps = [pltpu.async_copy(z_hbm.at[idx[a].at[sl]], rows[a], sem)
             for a in range(4)]
      for cp in cps:
        cp.wait()

      iot = lax.iota(jnp.int32, LANES)

      def lanesum(v):
        for sh in (8, 4, 2, 1):
          v = v + v.at[jnp.bitwise_xor(iot, sh)].get(
              mode="promise_in_bounds")
        return v

      @pl.loop(0, CP // LANES)
      def _grp(t):
        osl = pl.ds(g * CP + t * LANES, LANES)
        for o in range(2):
          red = jnp.zeros((LANES,), _f32)
          for kk in range(LANES):
            i = t * LANES + kk
            acc = (rows[2 * o][i, pl.ds(0, LANES)]
                   * rows[2 * o + 1][i, pl.ds(0, LANES)])
            for j in range(1, nseg):
              jl = pl.ds(j * LANES, LANES)
              acc = acc + rows[2 * o][i, jl] * rows[2 * o + 1][i, jl]
            red = jnp.where(iot == kk, lanesum(acc), red)
          outv[o][osl] = red

    pltpu.sync_copy(outv[0], pos_hbm.at[pl.ds(base, p_per)])
    pltpu.sync_copy(outv[1], neg_hbm.at[pl.ds(base, p_per)])

  return k


def _make_segsum(N, D, E):
  e_per = E // NW
  CH = 128
  n_full = e_per // CH
  ch_tail = e_per - n_full * CH
  nseg = 8
  stripe = (N // NS) // 8 * 8
  tail = N - NS * stripe
  mesh = plsc.VectorSubcoreMesh(core_axis_name="c", subcore_axis_name="s")

  @functools.partial(
      pl.kernel,
      mesh=mesh,
      out_type=(
          jax.ShapeDtypeStruct((NC, N, D), _f32),
          jax.ShapeDtypeStruct((NC, 1, N), _f32),
      ),
      scratch_types=[
          [pltpu.VMEM((CH,), jnp.int32)] * 2,
          [pltpu.VMEM((CH,), jnp.int32)] * 2,
          [pltpu.VMEM((CH,), _f32)] * 2,
          [pltpu.VMEM((CH, D), _f32)] * 2,
          pltpu.VMEM((N,), _f32),
          pltpu.VMEM_SHARED((N, D), _f32),
          pltpu.VMEM_SHARED((N,), _f32),
          pltpu.SemaphoreType.DMA,
          [pltpu.SemaphoreType.DMA] * 2,
      ],
  )
  def k(nfeat_hbm, src_hbm, dst_hbm, w_hbm, agg_hbm, ws_hbm,
        idx_s, idx_d, wv, rows, zv, acc_sh, ws_sh, gsem, ssem):
    c = lax.axis_index("c")
    s = lax.axis_index("s")
    wid = _wid(c, s)
    zeros16 = jnp.zeros((LANES,), _f32)

    @pl.loop(0, CH)
    def _zrows(i):
      for j in range(nseg):
        rows[0][i, pl.ds(j * LANES, LANES)] = zeros16

    @pl.loop(0, N // LANES)
    def _zv(i):
      zv[pl.ds(i * LANES, LANES)] = zeros16

    @pl.when(s == 0)
    def _zws():
      pltpu.sync_copy(zv, ws_sh)

    sbase = s * stripe
    full, rem = stripe // CH, stripe % CH
    for t in range(full):
      pltpu.sync_copy(rows[0], acc_sh.at[pl.ds(sbase + t * CH, CH)])
    if rem:
      pltpu.sync_copy(rows[0].at[pl.ds(0, rem)],
                      acc_sh.at[pl.ds(sbase + full * CH, rem)])
    if tail:
      @pl.when(s == NS - 1)
      def _ztail():
        pltpu.sync_copy(rows[0].at[pl.ds(0, tail)],
                        acc_sh.at[pl.ds(NS * stripe, tail)])
    plsc.subcore_barrier()

    ebase = wid * e_per

    def stage_process(base, b, ch):
      csl = pl.ds(0, ch)
      pltpu.sync_copy(src_hbm.at[pl.ds(base, ch)], idx_s[b].at[csl])
      cp_d = pltpu.async_copy(dst_hbm.at[pl.ds(base, ch)], idx_d[b].at[csl],
                              gsem)
      cp_w = pltpu.async_copy(w_hbm.at[pl.ds(base, ch)], wv[b].at[csl], gsem)
      cp_g = pltpu.async_copy(nfeat_hbm.at[idx_s[b].at[csl]],
                              rows[b].at[csl], gsem)
      cp_d.wait()
      cp_w.wait()
      cp_g.wait()

      @pl.loop(0, ch // LANES)
      def _scale(t):
        wv16 = wv[b][pl.ds(t * LANES, LANES)]
        for kk in range(LANES):
          wb = jnp.full((LANES,), wv16[kk], _f32)
          i = t * LANES + kk
          for j in range(nseg):
            sl = pl.ds(j * LANES, LANES)
            rows[b][i, sl] = rows[b][i, sl] * wb

      pltpu.async_copy(rows[b].at[csl], acc_sh.at[idx_d[b].at[csl]],
                       ssem[b], add=True)
      pltpu.async_copy(wv[b].at[csl], ws_sh.at[idx_d[b].at[csl]],
                       ssem[b], add=True)

    def drain(b, ch):
      csl = pl.ds(0, ch)
      pltpu.make_async_copy(rows[b].at[csl], acc_sh.at[idx_d[b].at[csl]],
                            ssem[b]).wait()
      pltpu.make_async_copy(wv[b].at[csl], ws_sh.at[idx_d[b].at[csl]],
                            ssem[b]).wait()

    n_pairs = n_full // 2

    @pl.loop(0, n_pairs)
    def _edges(kk):
      for b in range(2):
        @pl.when(kk > 0)
        def _d():
          drain(b, CH)
        stage_process(ebase + (2 * kk + b) * CH, b, CH)

    drain(0, CH)
    drain(1, CH)
    if ch_tail:
      stage_process(ebase + n_full * CH, 0, ch_tail)
      drain(0, ch_tail)

    plsc.subcore_barrier()
    pltpu.sync_copy(acc_sh.at[pl.ds(sbase, stripe)],
                    agg_hbm.at[c, pl.ds(sbase, stripe)])
    if tail:
      @pl.when(s == NS - 1)
      def _otail():
        pltpu.sync_copy(acc_sh.at[pl.ds(NS * stripe, tail)],
                        agg_hbm.at[c, pl.ds(NS * stripe, tail)])
    @pl.when(s == 0)
    def _ows():
      pltpu.sync_copy(ws_sh, ws_hbm.at[c, 0])

  return k


_BR = 2000


def _dot(a, b):
  return jnp.dot(a, b, preferred_element_type=_f32)


def _tc_proj_kernel(x_ref, wf_ref, bf_ref, emb_ref, qw_ref, qb_ref,
                    h_ref, nf_ref):
  h = _dot(x_ref[...], wf_ref[...]) + bf_ref[...][None, :] + emb_ref[...]
  h_ref[...] = h
  nf_ref[...] = jnp.maximum(_dot(h, qw_ref[...]) + qb_ref[...][None, :], 0.0)


def _tc_conv_kernel(aggp_ref, wsp_ref, hdst_ref, wa_ref, wb_ref, b_ref,
                    qw_ref, qb_ref, z_ref, nf_ref):
  agg = aggp_ref[0] + aggp_ref[1]
  ws = jnp.maximum(jnp.sum(wsp_ref[...], axis=1), 1.0)
  z = jnp.maximum(
      _dot(agg / ws[:, None], wa_ref[...]) + _dot(hdst_ref[...], wb_ref[...])
      + b_ref[...][None, :], 0.0)
  z_ref[...] = z
  nf_ref[...] = jnp.maximum(_dot(z, qw_ref[...]) + qb_ref[...][None, :], 0.0)


def _tc_final_kernel(aggp_ref, wsp_ref, hdst_ref, wa_ref, wb_ref, b_ref,
                     hitem_ref, z_ref):
  agg = aggp_ref[0] + aggp_ref[1]
  ws = jnp.maximum(jnp.sum(wsp_ref[...], axis=1), 1.0)
  z1 = jnp.maximum(
      _dot(agg / ws[:, None], wa_ref[...]) + _dot(hdst_ref[...], wb_ref[...])
      + b_ref[...][None, :], 0.0)
  z = hitem_ref[...] + z1
  nrm = jnp.sqrt(jnp.sum(z * z, axis=1, keepdims=True))
  nrm = jnp.where(nrm == 0.0, 1.0, nrm)
  z_ref[...] = z / nrm


def _tc_score_kernel(posr_ref, negr_ref,
                     bps_ref, bpd_ref, bns_ref, bnd_ref, out_ref):
  pos = posr_ref[...] + bps_ref[...] + bpd_ref[...]
  neg = negr_ref[...] + bns_ref[...] + bnd_ref[...]
  out_ref[...] = jnp.maximum(neg - pos + 1.0, 0.0)


def _row_spec(d):
  return pl.BlockSpec((_BR, d), lambda i: (i, 0))


def _full2_spec(a, b):
  return pl.BlockSpec((a, b), lambda i: (0, 0))


def _vec_spec(d):
  return pl.BlockSpec((d,), lambda i: (0,))


def kernel(x_feat, node_ids, edge_src0, edge_dst0, edge_w0,
           edge_src1, edge_dst1, edge_w1,
           pos_src, pos_dst, neg_src, neg_dst,
           W_feat, b_feat, emb_id,
           Q0w, Q0b, W0w, W0b, Q1w, Q1b, W1w, W1b, bias):
  N, D = x_feat.shape
  V = emb_id.shape[0]
  E = edge_src0.shape[0]
  EP = pos_src.shape[0]
  grid = (N // _BR,)

  def pad_idx(idx, tot):
    return jnp.concatenate(
        [idx.astype(jnp.int32), jnp.zeros((tot - idx.shape[0],), jnp.int32)])

  BN = 10240
  emb_rows = _make_gather_rows(V, D, BN)(emb_id, pad_idx(node_ids, BN))[:N]

  h_item, nf0 = pl.pallas_call(
      _tc_proj_kernel,
      grid=grid,
      in_specs=[_row_spec(D), _full2_spec(D, D), _vec_spec(D), _row_spec(D),
                _full2_spec(D, D), _vec_spec(D)],
      out_specs=[_row_spec(D), _row_spec(D)],
      out_shape=[jax.ShapeDtypeStruct((N, D), _f32)] * 2,
  )(x_feat, W_feat, b_feat, emb_rows, Q0w, Q0b)

  segsum = _make_segsum(N, D, E)

  aggp0, wsp0 = segsum(nf0, edge_src0.astype(jnp.int32),
                       edge_dst0.astype(jnp.int32), edge_w0)
  z0, nf1 = pl.pallas_call(
      _tc_conv_kernel,
      grid=grid,
      in_specs=[pl.BlockSpec((NC, _BR, D), lambda i: (0, i, 0)),
                pl.BlockSpec((_BR, NC), lambda i: (i, 0)),
                _row_spec(D), _full2_spec(D, D), _full2_spec(D, D),
                _vec_spec(D), _full2_spec(D, D), _vec_spec(D)],
      out_specs=[_row_spec(D), _row_spec(D)],
      out_shape=[jax.ShapeDtypeStruct((N, D), _f32)] * 2,
  )(aggp0, wsp0[:, 0, :].T, h_item, W0w[:D], W0w[D:], W0b, Q1w, Q1b)

  aggp1, wsp1 = segsum(nf1, edge_src1.astype(jnp.int32),
                       edge_dst1.astype(jnp.int32), edge_w1)
  z = pl.pallas_call(
      _tc_final_kernel,
      grid=grid,
      in_specs=[pl.BlockSpec((NC, _BR, D), lambda i: (0, i, 0)),
                pl.BlockSpec((_BR, NC), lambda i: (i, 0)),
                _row_spec(D), _full2_spec(D, D), _full2_spec(D, D),
                _vec_spec(D), _row_spec(D)],
      out_specs=_row_spec(D),
      out_shape=jax.ShapeDtypeStruct((N, D), _f32),
  )(aggp1, wsp1[:, 0, :].T, z0, W1w[:D], W1w[D:], W1b, h_item)

  EPP = 10240
  idx_all = jnp.concatenate([pad_idx(pos_src, EPP), pad_idx(pos_dst, EPP),
                             pad_idx(neg_src, EPP), pad_idx(neg_dst, EPP)])
  posr, negr = _make_score(N, D, EPP)(
      z, pad_idx(pos_src, EPP), pad_idx(pos_dst, EPP),
      pad_idx(neg_src, EPP), pad_idx(neg_dst, EPP))
  b_all = _make_gather_bias(N, V, 4 * EPP)(
      node_ids.astype(jnp.int32), bias, idx_all)

  sl = [slice(k * EPP, k * EPP + EP) for k in range(4)]
  out = pl.pallas_call(
      _tc_score_kernel,
      out_shape=jax.ShapeDtypeStruct((EP,), _f32),
  )(posr[:EP], negr[:EP],
    b_all[sl[0]], b_all[sl[1]], b_all[sl[2]], b_all[sl[3]])
  return out
